# Initial kernel scaffold; baseline (speedup 1.0000x reference)
import jax
import jax.numpy as jnp
from jax.experimental import pallas as pl


def kernel(train_indices_ref, train_embeddings_ref, train_indices_pos, train_embeddings_pos, indices, Z_ssps, embeddings, step_rel):
    raise NotImplementedError("write your pallas kernel here")


# speedup vs baseline: 8.2999x; 8.2999x over previous
import jax
import jax.numpy as jnp
from jax import lax
from jax.experimental import pallas as pl
from jax.experimental.pallas import tpu as pltpu

_B = 4096
_D = 128
_MBLK = 24
_PBLK = 16
_NB = 2
_GRID = _NB * _PBLK


def _body(step_ref,
          ti_ref_in, te_ref_in, tip_in, tep_in, idx2_in, z_in, emb_in,
          ti_ref_out, te_ref_out, tip_out, tep_out):
    i = pl.program_id(0)
    s = step_ref[0]

    @pl.when(jnp.logical_and(i < _MBLK, i != s))
    def _():
        te_ref_out[...] = te_ref_in[...]

    @pl.when(i == s)
    def _():
        te_ref_out[...] = z_in[...]

    j = lax.rem(i, _PBLK)
    ps = lax.rem(s, _PBLK)

    @pl.when(j != ps)
    def _():
        tep_out[...] = tep_in[...]

    @pl.when(j == ps)
    def _():
        tep_out[...] = emb_in[...]

    @pl.when(i == 0)
    def _():
        rows = _B // _D
        ti_ref_out[...] = ti_ref_in[...]
        ti_ref_out[pl.ds(s * rows, rows), :] = idx2_in[...]
        tip_out[...] = tip_in[...]
        tip_out[pl.ds(lax.rem(s * rows, tip_out.shape[0]), rows), :] = idx2_in[...]


def kernel(train_indices_ref, train_embeddings_ref, train_indices_pos,
           train_embeddings_pos, indices, Z_ssps, embeddings, step_rel):
    M = train_embeddings_ref.shape[0]
    P = train_indices_pos.shape[0]
    step = jnp.asarray(step_rel, jnp.int32).reshape(1)

    ti2 = train_indices_ref.reshape(M // _D, _D)
    tip2 = train_indices_pos.reshape(P // _D, _D)
    idx2 = indices.reshape(_B // _D, _D)

    full = lambda shape: pl.BlockSpec(shape, lambda i: (0,) * len(shape))

    out = pl.pallas_call(
        _body,
        grid=(_GRID,),
        in_specs=[
            pl.BlockSpec(memory_space=pltpu.MemorySpace.SMEM),
            full(ti2.shape),
            pl.BlockSpec((_B, _D), lambda i: (jnp.minimum(i, _MBLK - 1), 0)),
            full(tip2.shape),
            pl.BlockSpec((1, _B, _D), lambda i: (i // _PBLK, lax.rem(i, _PBLK), 0)),
            full(idx2.shape),
            full((_B, _D)),
            pl.BlockSpec((1, _B, _D), lambda i: (i // _PBLK, 0, 0)),
        ],
        out_specs=[
            full(ti2.shape),
            pl.BlockSpec((_B, _D), lambda i: (jnp.minimum(i, _MBLK - 1), 0)),
            full(tip2.shape),
            pl.BlockSpec((1, _B, _D), lambda i: (i // _PBLK, lax.rem(i, _PBLK), 0)),
        ],
        out_shape=[
            jax.ShapeDtypeStruct(ti2.shape, jnp.int32),
            jax.ShapeDtypeStruct((M, _D), jnp.float32),
            jax.ShapeDtypeStruct(tip2.shape, jnp.int32),
            jax.ShapeDtypeStruct((_NB, P, _D), jnp.float32),
        ],
        compiler_params=pltpu.CompilerParams(
            dimension_semantics=("arbitrary",),
        ),
    )(step, ti2, train_embeddings_ref, tip2, train_embeddings_pos, idx2,
      Z_ssps, embeddings)

    return (out[0].reshape(M), out[1], out[2].reshape(P), out[3])


# speedup vs baseline: 8.3604x; 1.0073x over previous
import jax
import jax.numpy as jnp
from jax import lax
from jax.experimental import pallas as pl
from jax.experimental.pallas import tpu as pltpu

_B = 4096
_D = 128
_MBLK = 24
_PBLK = 16
_NB = 2
_GRID = _NB * _PBLK


def _body(step_ref,
          ti_ref_in, te_ref_in, tip_in, tep_in, idx2_in, z_in, emb_in,
          ti_ref_out, te_ref_out, tip_out, tep_out):
    i = pl.program_id(0)
    s = step_ref[0]

    @pl.when(jnp.logical_and(i < _MBLK, i != s))
    def _():
        te_ref_out[...] = te_ref_in[...]

    @pl.when(i == s)
    def _():
        te_ref_out[...] = z_in[...]

    j = lax.rem(i, _PBLK)
    ps = lax.rem(s, _PBLK)

    @pl.when(j != ps)
    def _():
        tep_out[...] = tep_in[...]

    @pl.when(j == ps)
    def _():
        tep_out[...] = emb_in[...]

    @pl.when(i == 0)
    def _():
        rows = _B // _D
        ti_ref_out[...] = ti_ref_in[...]
        ti_ref_out[pl.ds(s * rows, rows), :] = idx2_in[...]
        tip_out[...] = tip_in[...]
        tip_out[pl.ds(lax.rem(s * rows, tip_out.shape[0]), rows), :] = idx2_in[...]


def kernel(train_indices_ref, train_embeddings_ref, train_indices_pos,
           train_embeddings_pos, indices, Z_ssps, embeddings, step_rel):
    M = train_embeddings_ref.shape[0]
    P = train_indices_pos.shape[0]
    step = jnp.asarray(step_rel, jnp.int32).reshape(1)

    ti2 = train_indices_ref.reshape(M // _D, _D)
    tip2 = train_indices_pos.reshape(P // _D, _D)
    idx2 = indices.reshape(_B // _D, _D)

    full = lambda shape: pl.BlockSpec(shape, lambda i, s: (0,) * len(shape))

    def _ref_in_idx(i, s):
        skip = jnp.maximum(s[0] - 1, 0)
        return (jnp.where(i == s[0], skip, jnp.minimum(i, _MBLK - 1)), 0)

    def _pos_in_idx(i, s):
        j = lax.rem(i, _PBLK)
        ps = lax.rem(s[0], _PBLK)
        skip = jnp.maximum(ps - 1, 0)
        return (i // _PBLK, jnp.where(j == ps, skip, j), 0)

    out = pl.pallas_call(
        _body,
        grid_spec=pltpu.PrefetchScalarGridSpec(
            num_scalar_prefetch=1,
            grid=(_GRID,),
            in_specs=[
                full(ti2.shape),
                pl.BlockSpec((_B, _D), _ref_in_idx),
                full(tip2.shape),
                pl.BlockSpec((1, _B, _D), _pos_in_idx),
                full(idx2.shape),
                full((_B, _D)),
                pl.BlockSpec((1, _B, _D), lambda i, s: (i // _PBLK, 0, 0)),
            ],
            out_specs=[
                full(ti2.shape),
                pl.BlockSpec((_B, _D),
                             lambda i, s: (jnp.minimum(i, _MBLK - 1), 0)),
                full(tip2.shape),
                pl.BlockSpec((1, _B, _D),
                             lambda i, s: (i // _PBLK, lax.rem(i, _PBLK), 0)),
            ],
        ),
        out_shape=[
            jax.ShapeDtypeStruct(ti2.shape, jnp.int32),
            jax.ShapeDtypeStruct((M, _D), jnp.float32),
            jax.ShapeDtypeStruct(tip2.shape, jnp.int32),
            jax.ShapeDtypeStruct((_NB, P, _D), jnp.float32),
        ],
        compiler_params=pltpu.CompilerParams(
            dimension_semantics=("arbitrary",),
        ),
    )(step, ti2, train_embeddings_ref, tip2, train_embeddings_pos, idx2,
      Z_ssps, embeddings)

    return (out[0].reshape(M), out[1], out[2].reshape(P), out[3])
